# baseline (speedup 1.0000x reference)
import functools

import jax
import jax.numpy as jnp
from jax import lax
from jax.experimental import pallas as pl
from jax.experimental.pallas import tpu as pltpu
from jax.experimental.pallas import tpu_sc as plsc

N = 10000
E = 320000
D_IN = 128
D_H = 64
D_OUT = 16
SLOPE_NEG = 0.01

NPAD = 10240
ROWS_PER_TILE = NPAD // 16
NC = 2
NS = 16
KB = 128
BPT = 80
NBLK = NC * NS * BPT
E2 = NBLK * KB
NBUF = 8
NG = BPT // NBUF
_RING_CFG = {64: (8, 6), 16: (10, 8)}


def _mesh():
    return plsc.VectorSubcoreMesh(core_axis_name="c", subcore_axis_name="s")


def _degree_body(dst_hbm, out_hbm, didx, ones_v, zbuf, acc, dsem):
    cid = lax.axis_index("c")
    sid = lax.axis_index("s")
    wid = cid * NS + sid
    base_row = sid * ROWS_PER_TILE

    ov = jnp.ones((16,), jnp.float32)
    zv = jnp.zeros((16,), jnp.float32)
    for i in range(KB // 16):
        ones_v[pl.ds(i * 16, 16)] = ov
    for i in range(ROWS_PER_TILE // 16):
        zbuf[pl.ds(i * 16, 16)] = zv
    pltpu.sync_copy(dst_hbm.at[pl.ds(wid * BPT, BPT)], didx)
    pltpu.sync_copy(zbuf, acc.at[pl.ds(base_row, ROWS_PER_TILE)])
    plsc.subcore_barrier()

    @pl.loop(0, BPT)
    def _(j):
        pltpu.async_copy(ones_v, acc.at[didx.at[j]], dsem, add=True)

    @pl.loop(0, BPT)
    def _(j):
        pltpu.make_async_copy(ones_v, acc.at[pl.ds(0, KB)], dsem).wait()

    plsc.subcore_barrier()
    pltpu.sync_copy(acc.at[pl.ds(base_row, ROWS_PER_TILE)],
                    out_hbm.at[cid, pl.ds(base_row, ROWS_PER_TILE)])


_degree = pl.kernel(
    _degree_body,
    out_type=jax.ShapeDtypeStruct((NC, NPAD), jnp.float32),
    mesh=_mesh(),
    compiler_params=pltpu.CompilerParams(use_tc_tiling_on_sc=False),
    scratch_types=[
        pltpu.VMEM((BPT, KB), jnp.int32),
        pltpu.VMEM((KB,), jnp.float32),
        pltpu.VMEM((ROWS_PER_TILE,), jnp.float32),
        pltpu.VMEM_SHARED((NPAD,), jnp.float32),
        pltpu.SemaphoreType.DMA,
    ],
)


def _agg_body(F, g_hbm, src_hbm, dst_hbm, out_hbm, sidx, didx, rows, ztile,
              acc, gsem, ssem, zsem):
    RING, LOOK = _RING_CFG[F]
    cid = lax.axis_index("c")
    sid = lax.axis_index("s")
    wid = cid * NS + sid
    base_row = sid * ROWS_PER_TILE

    zv = jnp.zeros((16,), jnp.float32)
    for r in range(16):
        for c in range(F // 16):
            ztile[r, pl.ds(c * 16, 16)] = zv

    nz = ROWS_PER_TILE // 16

    @pl.loop(0, nz)
    def _(r):
        pltpu.async_copy(ztile, acc.at[pl.ds(base_row + r * 16, 16)], zsem)

    pltpu.sync_copy(src_hbm.at[pl.ds(wid * BPT, BPT)], sidx)
    pltpu.sync_copy(dst_hbm.at[pl.ds(wid * BPT, BPT)], didx)

    def gwait(b):
        pltpu.make_async_copy(g_hbm.at[sidx.at[0]], rows.at[b],
                              gsem.at[b]).wait()

    def swait(b):
        pltpu.make_async_copy(rows.at[b], acc.at[didx.at[0]],
                              ssem.at[b]).wait()

    @pl.loop(0, nz)
    def _(r):
        pltpu.make_async_copy(ztile, acc.at[pl.ds(base_row, 16)], zsem).wait()

    plsc.subcore_barrier()

    for c in range(LOOK):
        pltpu.async_copy(g_hbm.at[sidx.at[c]], rows.at[c], gsem.at[c])

    def visit(c, b, do_swait, do_refill):
        gwait(b)
        pltpu.async_copy(rows.at[b], acc.at[didx.at[c]],
                         ssem.at[b], add=True)
        if do_refill:
            b2 = (b + LOOK) % RING
            if do_swait:
                swait(b2)
            pltpu.async_copy(g_hbm.at[sidx.at[c + LOOK]], rows.at[b2],
                             gsem.at[b2])

    for b in range(RING):
        visit(b, b, b >= RING - LOOK, True)

    @pl.loop(1, BPT // RING - 1)
    def _(g):
        c0 = g * RING
        for b in range(RING):
            visit(c0 + b, b, True, True)

    for b in range(RING):
        visit(BPT - RING + b, b, True, b < RING - LOOK)
    for b in range(RING):
        swait(b)

    plsc.subcore_barrier()
    pltpu.sync_copy(acc.at[pl.ds(base_row, ROWS_PER_TILE)],
                    out_hbm.at[cid, pl.ds(base_row, ROWS_PER_TILE)])


def _make_agg(F):
    RING, _ = _RING_CFG[F]
    return pl.kernel(
        functools.partial(_agg_body, F),
        out_type=jax.ShapeDtypeStruct((NC, NPAD, F), jnp.float32),
        mesh=_mesh(),
        compiler_params=pltpu.CompilerParams(use_tc_tiling_on_sc=False),
        scratch_types=[
            pltpu.VMEM((BPT, KB), jnp.int32),
            pltpu.VMEM((BPT, KB), jnp.int32),
            pltpu.VMEM((RING, KB, F), jnp.float32),
            pltpu.VMEM((16, F), jnp.float32),
            pltpu.VMEM_SHARED((NPAD, F), jnp.float32),
            pltpu.SemaphoreType.DMA((RING,)),
            pltpu.SemaphoreType.DMA((RING,)),
            pltpu.SemaphoreType.DMA,
        ],
    )


_agg64 = _make_agg(D_H)
_agg16 = _make_agg(D_OUT)


GRID = 10
RB = N // GRID
RBP = NPAD // GRID


def _tc_first(x_ref, w_ref, d0_ref, d1_ref, g_ref, dinv_ref):
    deg = d0_ref[...] + d1_ref[...] + 1.0
    dinv = lax.rsqrt(deg)
    dinv_ref[...] = dinv
    h = jnp.dot(x_ref[...], w_ref[...], preferred_element_type=jnp.float32)
    g_ref[pl.ds(0, RB), :] = h * dinv[0:RB]


def _tc_mid(p_ref, g1_ref, dinv_ref, b1_ref, w2_ref, g2_ref):
    dinv = dinv_ref[...]
    t = (p_ref[0] + p_ref[1] + g1_ref[...]) * dinv + b1_ref[...]
    z = jnp.where(t >= 0.0, t, SLOPE_NEG * t)
    g2_ref[...] = jnp.dot(z, w2_ref[...],
                          preferred_element_type=jnp.float32) * dinv


def _tc_final(q_ref, g2_ref, dinv_ref, b2_ref, o_ref):
    full = ((q_ref[0] + q_ref[1] + g2_ref[...]) * dinv_ref[...]
            + b2_ref[...])
    o_ref[...] = full[0:RB, :]


def kernel(x, edge_index, W1, b1, W2, b2):
    pad = N + (jnp.arange(E2 - E, dtype=jnp.int32) % (NPAD - N))
    src2 = jnp.concatenate([edge_index[0], pad]).reshape(NBLK, KB)
    dst2 = jnp.concatenate([edge_index[1], pad]).reshape(NBLK, KB)

    degp = _degree(dst2)

    g1, dinv = pl.pallas_call(
        _tc_first,
        grid=(GRID,),
        in_specs=[
            pl.BlockSpec((RB, D_IN), lambda i: (i, 0)),
            pl.BlockSpec((D_IN, D_H), lambda i: (0, 0)),
            pl.BlockSpec((RBP, 1), lambda i: (i, 0)),
            pl.BlockSpec((RBP, 1), lambda i: (i, 0)),
        ],
        out_specs=(pl.BlockSpec((RBP, D_H), lambda i: (i, 0)),
                   pl.BlockSpec((RBP, 1), lambda i: (i, 0))),
        out_shape=(jax.ShapeDtypeStruct((NPAD, D_H), jnp.float32),
                   jax.ShapeDtypeStruct((NPAD, 1), jnp.float32)),
    )(x, W1, degp[0].reshape(NPAD, 1), degp[1].reshape(NPAD, 1))

    p = _agg64(g1, src2, dst2)

    g2 = pl.pallas_call(
        _tc_mid,
        grid=(GRID,),
        in_specs=[
            pl.BlockSpec((2, RBP, D_H), lambda i: (0, i, 0)),
            pl.BlockSpec((RBP, D_H), lambda i: (i, 0)),
            pl.BlockSpec((RBP, 1), lambda i: (i, 0)),
            pl.BlockSpec((1, D_H), lambda i: (0, 0)),
            pl.BlockSpec((D_H, D_OUT), lambda i: (0, 0)),
        ],
        out_specs=pl.BlockSpec((RBP, D_OUT), lambda i: (i, 0)),
        out_shape=jax.ShapeDtypeStruct((NPAD, D_OUT), jnp.float32),
    )(p, g1, dinv, b1.reshape(1, D_H), W2)

    q = _agg16(g2, src2, dst2)

    return pl.pallas_call(
        _tc_final,
        grid=(GRID,),
        in_specs=[
            pl.BlockSpec((2, RBP, D_OUT), lambda i: (0, i, 0)),
            pl.BlockSpec((RBP, D_OUT), lambda i: (i, 0)),
            pl.BlockSpec((RBP, 1), lambda i: (i, 0)),
            pl.BlockSpec((1, D_OUT), lambda i: (0, 0)),
        ],
        out_specs=pl.BlockSpec((RB, D_OUT), lambda i: (i, 0)),
        out_shape=jax.ShapeDtypeStruct((N, D_OUT), jnp.float32),
    )(q, g2, dinv, b2.reshape(1, D_OUT))


# speedup vs baseline: 1.0036x; 1.0036x over previous
import functools

import jax
import jax.numpy as jnp
from jax import lax
from jax.experimental import pallas as pl
from jax.experimental.pallas import tpu as pltpu
from jax.experimental.pallas import tpu_sc as plsc

N = 10000
E = 320000
D_IN = 128
D_H = 64
D_OUT = 16
SLOPE_NEG = 0.01

NPAD = 10240
ROWS_PER_TILE = NPAD // 16
NC = 2
NS = 16
KB = 128
BPT = 80
NBLK = NC * NS * BPT
E2 = NBLK * KB
NBUF = 8
NG = BPT // NBUF
_RING_CFG = {64: (8, 6), 16: (10, 8)}


def _mesh():
    return plsc.VectorSubcoreMesh(core_axis_name="c", subcore_axis_name="s")


def _degree_body(dst_hbm, out_hbm, didx, ones_v, zbuf, acc, dsem):
    cid = lax.axis_index("c")
    sid = lax.axis_index("s")
    wid = cid * NS + sid
    base_row = sid * ROWS_PER_TILE

    ov = jnp.ones((16,), jnp.float32)
    zv = jnp.zeros((16,), jnp.float32)
    for i in range(KB // 16):
        ones_v[pl.ds(i * 16, 16)] = ov
    for i in range(ROWS_PER_TILE // 16):
        zbuf[pl.ds(i * 16, 16)] = zv
    pltpu.sync_copy(dst_hbm.at[pl.ds(wid * BPT, BPT)], didx)
    pltpu.sync_copy(zbuf, acc.at[pl.ds(base_row, ROWS_PER_TILE)])
    plsc.subcore_barrier()

    @pl.loop(0, BPT)
    def _(j):
        pltpu.async_copy(ones_v, acc.at[didx.at[j]], dsem, add=True)

    @pl.loop(0, BPT)
    def _(j):
        pltpu.make_async_copy(ones_v, acc.at[pl.ds(0, KB)], dsem).wait()

    plsc.subcore_barrier()
    pltpu.sync_copy(acc.at[pl.ds(base_row, ROWS_PER_TILE)],
                    out_hbm.at[cid, pl.ds(base_row, ROWS_PER_TILE)])


_degree = pl.kernel(
    _degree_body,
    out_type=jax.ShapeDtypeStruct((NC, NPAD), jnp.float32),
    mesh=_mesh(),
    compiler_params=pltpu.CompilerParams(use_tc_tiling_on_sc=False),
    scratch_types=[
        pltpu.VMEM((BPT, KB), jnp.int32),
        pltpu.VMEM((KB,), jnp.float32),
        pltpu.VMEM((ROWS_PER_TILE,), jnp.float32),
        pltpu.VMEM_SHARED((NPAD,), jnp.float32),
        pltpu.SemaphoreType.DMA,
    ],
)


def _agg_body(F, g_hbm, src_hbm, dst_hbm, out_hbm, sidx, didx, rows, ztile,
              acc, gsem, ssem, zsem):
    RING, LOOK = _RING_CFG[F]
    cid = lax.axis_index("c")
    sid = lax.axis_index("s")
    wid = cid * NS + sid
    base_row = sid * ROWS_PER_TILE

    zv = jnp.zeros((16,), jnp.float32)
    for r in range(16):
        for c in range(F // 16):
            ztile[r, pl.ds(c * 16, 16)] = zv

    nz = ROWS_PER_TILE // 16

    @pl.loop(0, nz)
    def _(r):
        pltpu.async_copy(ztile, acc.at[pl.ds(base_row + r * 16, 16)], zsem)

    pltpu.sync_copy(src_hbm.at[pl.ds(wid * BPT, BPT)], sidx)
    pltpu.sync_copy(dst_hbm.at[pl.ds(wid * BPT, BPT)], didx)

    def gwait(b):
        pltpu.make_async_copy(g_hbm.at[sidx.at[0]], rows.at[b],
                              gsem.at[b]).wait()

    def swait(b):
        pltpu.make_async_copy(rows.at[b], acc.at[didx.at[0]],
                              ssem.at[b]).wait()

    @pl.loop(0, nz)
    def _(r):
        pltpu.make_async_copy(ztile, acc.at[pl.ds(base_row, 16)], zsem).wait()

    plsc.subcore_barrier()

    for c in range(LOOK):
        pltpu.async_copy(g_hbm.at[sidx.at[c]], rows.at[c], gsem.at[c])

    def visit(c, b, do_swait, do_refill):
        gwait(b)
        pltpu.async_copy(rows.at[b], acc.at[didx.at[c]],
                         ssem.at[b], add=True)
        if do_refill:
            b2 = (b + LOOK) % RING
            if do_swait:
                swait(b2)
            pltpu.async_copy(g_hbm.at[sidx.at[c + LOOK]], rows.at[b2],
                             gsem.at[b2])

    for b in range(RING):
        visit(b, b, b >= RING - LOOK, True)

    @pl.loop(1, BPT // RING - 1)
    def _(g):
        c0 = g * RING
        for b in range(RING):
            visit(c0 + b, b, True, True)

    for b in range(RING):
        visit(BPT - RING + b, b, True, b < RING - LOOK)
    for b in range(RING):
        swait(b)

    plsc.subcore_barrier()
    pltpu.sync_copy(acc.at[pl.ds(base_row, ROWS_PER_TILE)],
                    out_hbm.at[cid, pl.ds(base_row, ROWS_PER_TILE)])


def _make_agg(F):
    RING, _ = _RING_CFG[F]
    return pl.kernel(
        functools.partial(_agg_body, F),
        out_type=jax.ShapeDtypeStruct((NC, NPAD, F), jnp.float32),
        mesh=_mesh(),
        compiler_params=pltpu.CompilerParams(use_tc_tiling_on_sc=False),
        scratch_types=[
            pltpu.VMEM((BPT, KB), jnp.int32),
            pltpu.VMEM((BPT, KB), jnp.int32),
            pltpu.VMEM((RING, KB, F), jnp.float32),
            pltpu.VMEM((16, F), jnp.float32),
            pltpu.VMEM_SHARED((NPAD, F), jnp.float32),
            pltpu.SemaphoreType.DMA((RING,)),
            pltpu.SemaphoreType.DMA((RING,)),
            pltpu.SemaphoreType.DMA,
        ],
    )


_agg64 = _make_agg(D_H)
_agg16 = _make_agg(D_OUT)


GRID = 10
RBP = NPAD // GRID


def _tc_first(x_ref, w_ref, d0_ref, d1_ref, g_ref, dinv_ref):
    deg = d0_ref[...] + d1_ref[...] + 1.0
    dinv = lax.rsqrt(deg)
    dinv_ref[...] = dinv
    h = jnp.dot(x_ref[...], w_ref[...], preferred_element_type=jnp.float32)
    g_ref[...] = h * dinv


def _tc_mid(p_ref, g1_ref, dinv_ref, b1_ref, w2_ref, g2_ref):
    dinv = dinv_ref[...]
    t = (p_ref[0] + p_ref[1] + g1_ref[...]) * dinv + b1_ref[...]
    z = jnp.where(t >= 0.0, t, SLOPE_NEG * t)
    g2_ref[...] = jnp.dot(z, w2_ref[...],
                          preferred_element_type=jnp.float32) * dinv


def _tc_final(q_ref, g2_ref, dinv_ref, b2_ref, o_ref):
    o_ref[...] = ((q_ref[0] + q_ref[1] + g2_ref[...]) * dinv_ref[...]
                  + b2_ref[...])


def kernel(x, edge_index, W1, b1, W2, b2):
    pad = N + (jnp.arange(E2 - E, dtype=jnp.int32) % (NPAD - N))
    src2 = jnp.concatenate([edge_index[0], pad]).reshape(NBLK, KB)
    dst2 = jnp.concatenate([edge_index[1], pad]).reshape(NBLK, KB)

    degp = _degree(dst2)

    g1, dinv = pl.pallas_call(
        _tc_first,
        grid=(GRID,),
        in_specs=[
            pl.BlockSpec((RBP, D_IN), lambda i: (i, 0)),
            pl.BlockSpec((D_IN, D_H), lambda i: (0, 0)),
            pl.BlockSpec((RBP, 1), lambda i: (i, 0)),
            pl.BlockSpec((RBP, 1), lambda i: (i, 0)),
        ],
        out_specs=(pl.BlockSpec((RBP, D_H), lambda i: (i, 0)),
                   pl.BlockSpec((RBP, 1), lambda i: (i, 0))),
        out_shape=(jax.ShapeDtypeStruct((NPAD, D_H), jnp.float32),
                   jax.ShapeDtypeStruct((NPAD, 1), jnp.float32)),
    )(x, W1, degp[0].reshape(NPAD, 1), degp[1].reshape(NPAD, 1))

    p = _agg64(g1, src2, dst2)

    g2 = pl.pallas_call(
        _tc_mid,
        grid=(GRID,),
        in_specs=[
            pl.BlockSpec((2, RBP, D_H), lambda i: (0, i, 0)),
            pl.BlockSpec((RBP, D_H), lambda i: (i, 0)),
            pl.BlockSpec((RBP, 1), lambda i: (i, 0)),
            pl.BlockSpec((1, D_H), lambda i: (0, 0)),
            pl.BlockSpec((D_H, D_OUT), lambda i: (0, 0)),
        ],
        out_specs=pl.BlockSpec((RBP, D_OUT), lambda i: (i, 0)),
        out_shape=jax.ShapeDtypeStruct((NPAD, D_OUT), jnp.float32),
    )(p, g1, dinv, b1.reshape(1, D_H), W2)

    q = _agg16(g2, src2, dst2)

    return pl.pallas_call(
        _tc_final,
        grid=(GRID,),
        in_specs=[
            pl.BlockSpec((2, RBP, D_OUT), lambda i: (0, i, 0)),
            pl.BlockSpec((RBP, D_OUT), lambda i: (i, 0)),
            pl.BlockSpec((RBP, 1), lambda i: (i, 0)),
            pl.BlockSpec((1, D_OUT), lambda i: (0, 0)),
        ],
        out_specs=pl.BlockSpec((RBP, D_OUT), lambda i: (i, 0)),
        out_shape=jax.ShapeDtypeStruct((N, D_OUT), jnp.float32),
    )(q, g2, dinv, b2.reshape(1, D_OUT))


# speedup vs baseline: 1.0139x; 1.0103x over previous
import functools

import jax
import jax.numpy as jnp
from jax import lax
from jax.experimental import pallas as pl
from jax.experimental.pallas import tpu as pltpu
from jax.experimental.pallas import tpu_sc as plsc

N = 10000
E = 320000
D_IN = 128
D_H = 64
D_OUT = 16
SLOPE_NEG = 0.01

NPAD = 10240
ROWS_PER_TILE = NPAD // 16
NC = 2
NS = 16
KB = 128
BPT = 80
NBLK = E // KB
TAIL = NBLK - (NC * NS - 1) * BPT
NBUF = 8
NG = BPT // NBUF
_RING_CFG = {64: (8, 6), 16: (10, 8)}


def _mesh():
    return plsc.VectorSubcoreMesh(core_axis_name="c", subcore_axis_name="s")


def _degree_body(dst_hbm, out_hbm, didx, ones_v, zbuf, acc, dsem):
    cid = lax.axis_index("c")
    sid = lax.axis_index("s")
    wid = cid * NS + sid
    base_row = sid * ROWS_PER_TILE

    ov = jnp.ones((16,), jnp.float32)
    zv = jnp.zeros((16,), jnp.float32)
    for i in range(KB // 16):
        ones_v[pl.ds(i * 16, 16)] = ov
    for i in range(ROWS_PER_TILE // 16):
        zbuf[pl.ds(i * 16, 16)] = zv
    base_blk = wid * BPT
    nblk = jnp.minimum(BPT, NBLK - base_blk)

    @pl.when(base_blk + BPT <= NBLK)
    def _():
        pltpu.sync_copy(dst_hbm.at[pl.ds(base_blk, BPT)], didx)

    @pl.when(base_blk + BPT > NBLK)
    def _():
        pltpu.sync_copy(dst_hbm.at[pl.ds(base_blk, TAIL)],
                        didx.at[pl.ds(0, TAIL)])

    pltpu.sync_copy(zbuf, acc.at[pl.ds(base_row, ROWS_PER_TILE)])
    plsc.subcore_barrier()

    @pl.loop(0, nblk)
    def _(j):
        pltpu.async_copy(ones_v, acc.at[didx.at[j]], dsem, add=True)

    @pl.loop(0, nblk)
    def _(j):
        pltpu.make_async_copy(ones_v, acc.at[pl.ds(0, KB)], dsem).wait()

    plsc.subcore_barrier()
    pltpu.sync_copy(acc.at[pl.ds(base_row, ROWS_PER_TILE)],
                    out_hbm.at[cid, pl.ds(base_row, ROWS_PER_TILE)])


_degree = pl.kernel(
    _degree_body,
    out_type=jax.ShapeDtypeStruct((NC, NPAD), jnp.float32),
    mesh=_mesh(),
    compiler_params=pltpu.CompilerParams(use_tc_tiling_on_sc=False),
    scratch_types=[
        pltpu.VMEM((BPT, KB), jnp.int32),
        pltpu.VMEM((KB,), jnp.float32),
        pltpu.VMEM((ROWS_PER_TILE,), jnp.float32),
        pltpu.VMEM_SHARED((NPAD,), jnp.float32),
        pltpu.SemaphoreType.DMA,
    ],
)


def _agg_body(F, g_hbm, src_hbm, dst_hbm, out_hbm, sidx, didx, rows, ztile,
              acc, gsem, ssem, zsem):
    RING, LOOK = _RING_CFG[F]
    cid = lax.axis_index("c")
    sid = lax.axis_index("s")
    wid = cid * NS + sid
    base_row = sid * ROWS_PER_TILE

    zv = jnp.zeros((16,), jnp.float32)
    for r in range(16):
        for c in range(F // 16):
            ztile[r, pl.ds(c * 16, 16)] = zv

    nz = ROWS_PER_TILE // 16

    @pl.loop(0, nz)
    def _(r):
        pltpu.async_copy(ztile, acc.at[pl.ds(base_row + r * 16, 16)], zsem)

    base_blk = wid * BPT

    @pl.when(base_blk + BPT <= NBLK)
    def _():
        pltpu.sync_copy(src_hbm.at[pl.ds(base_blk, BPT)], sidx)
        pltpu.sync_copy(dst_hbm.at[pl.ds(base_blk, BPT)], didx)

    @pl.when(base_blk + BPT > NBLK)
    def _():
        pltpu.sync_copy(src_hbm.at[pl.ds(base_blk, TAIL)],
                        sidx.at[pl.ds(0, TAIL)])
        pltpu.sync_copy(dst_hbm.at[pl.ds(base_blk, TAIL)],
                        didx.at[pl.ds(0, TAIL)])

    def gwait(b):
        pltpu.make_async_copy(g_hbm.at[sidx.at[0]], rows.at[b],
                              gsem.at[b]).wait()

    def swait(b):
        pltpu.make_async_copy(rows.at[b], acc.at[didx.at[0]],
                              ssem.at[b]).wait()

    @pl.loop(0, nz)
    def _(r):
        pltpu.make_async_copy(ztile, acc.at[pl.ds(base_row, 16)], zsem).wait()

    plsc.subcore_barrier()

    for c in range(LOOK):
        @pl.when(base_blk + c < NBLK)
        def _(c=c):
            pltpu.async_copy(g_hbm.at[sidx.at[c]], rows.at[c], gsem.at[c])

    def visit(c, b, do_swait, do_refill):
        @pl.when(base_blk + c < NBLK)
        def _():
            gwait(b)
            pltpu.async_copy(rows.at[b], acc.at[didx.at[c]],
                             ssem.at[b], add=True)
        if do_refill:
            b2 = (b + LOOK) % RING

            @pl.when(base_blk + c + LOOK < NBLK)
            def _():
                if do_swait:
                    swait(b2)
                pltpu.async_copy(g_hbm.at[sidx.at[c + LOOK]], rows.at[b2],
                                 gsem.at[b2])

    for b in range(RING):
        visit(b, b, b >= RING - LOOK, True)

    @pl.loop(1, BPT // RING - 1)
    def _(g):
        c0 = g * RING
        for b in range(RING):
            visit(c0 + b, b, True, True)

    for b in range(RING):
        visit(BPT - RING + b, b, True, b < RING - LOOK)
    for b in range(RING):
        swait(b)

    plsc.subcore_barrier()
    pltpu.sync_copy(acc.at[pl.ds(base_row, ROWS_PER_TILE)],
                    out_hbm.at[cid, pl.ds(base_row, ROWS_PER_TILE)])


def _make_agg(F):
    RING, _ = _RING_CFG[F]
    return pl.kernel(
        functools.partial(_agg_body, F),
        out_type=jax.ShapeDtypeStruct((NC, NPAD, F), jnp.float32),
        mesh=_mesh(),
        compiler_params=pltpu.CompilerParams(use_tc_tiling_on_sc=False),
        scratch_types=[
            pltpu.VMEM((BPT, KB), jnp.int32),
            pltpu.VMEM((BPT, KB), jnp.int32),
            pltpu.VMEM((RING, KB, F), jnp.float32),
            pltpu.VMEM((16, F), jnp.float32),
            pltpu.VMEM_SHARED((NPAD, F), jnp.float32),
            pltpu.SemaphoreType.DMA((RING,)),
            pltpu.SemaphoreType.DMA((RING,)),
            pltpu.SemaphoreType.DMA,
        ],
    )


_agg64 = _make_agg(D_H)
_agg16 = _make_agg(D_OUT)


GRID = 10
RBP = NPAD // GRID


def _tc_first(x_ref, w_ref, d0_ref, d1_ref, g_ref, dinv_ref):
    deg = d0_ref[...] + d1_ref[...] + 1.0
    dinv = lax.rsqrt(deg)
    dinv_ref[...] = dinv
    h = jnp.dot(x_ref[...], w_ref[...], preferred_element_type=jnp.float32)
    g_ref[...] = h * dinv


def _tc_mid(p_ref, g1_ref, dinv_ref, b1_ref, w2_ref, g2_ref):
    dinv = dinv_ref[...]
    t = (p_ref[0] + p_ref[1] + g1_ref[...]) * dinv + b1_ref[...]
    z = jnp.where(t >= 0.0, t, SLOPE_NEG * t)
    g2_ref[...] = jnp.dot(z, w2_ref[...],
                          preferred_element_type=jnp.float32) * dinv


def _tc_final(q_ref, g2_ref, dinv_ref, b2_ref, o_ref):
    o_ref[...] = ((q_ref[0] + q_ref[1] + g2_ref[...]) * dinv_ref[...]
                  + b2_ref[...])


def kernel(x, edge_index, W1, b1, W2, b2):
    src2 = edge_index[0].reshape(NBLK, KB)
    dst2 = edge_index[1].reshape(NBLK, KB)

    degp = _degree(dst2)

    g1, dinv = pl.pallas_call(
        _tc_first,
        grid=(GRID,),
        in_specs=[
            pl.BlockSpec((RBP, D_IN), lambda i: (i, 0)),
            pl.BlockSpec((D_IN, D_H), lambda i: (0, 0)),
            pl.BlockSpec((RBP, 1), lambda i: (i, 0)),
            pl.BlockSpec((RBP, 1), lambda i: (i, 0)),
        ],
        out_specs=(pl.BlockSpec((RBP, D_H), lambda i: (i, 0)),
                   pl.BlockSpec((RBP, 1), lambda i: (i, 0))),
        out_shape=(jax.ShapeDtypeStruct((NPAD, D_H), jnp.float32),
                   jax.ShapeDtypeStruct((NPAD, 1), jnp.float32)),
    )(x, W1, degp[0].reshape(NPAD, 1), degp[1].reshape(NPAD, 1))

    p = _agg64(g1, src2, dst2)

    g2 = pl.pallas_call(
        _tc_mid,
        grid=(GRID,),
        in_specs=[
            pl.BlockSpec((2, RBP, D_H), lambda i: (0, i, 0)),
            pl.BlockSpec((RBP, D_H), lambda i: (i, 0)),
            pl.BlockSpec((RBP, 1), lambda i: (i, 0)),
            pl.BlockSpec((1, D_H), lambda i: (0, 0)),
            pl.BlockSpec((D_H, D_OUT), lambda i: (0, 0)),
        ],
        out_specs=pl.BlockSpec((RBP, D_OUT), lambda i: (i, 0)),
        out_shape=jax.ShapeDtypeStruct((NPAD, D_OUT), jnp.float32),
    )(p, g1, dinv, b1.reshape(1, D_H), W2)

    q = _agg16(g2, src2, dst2)

    return pl.pallas_call(
        _tc_final,
        grid=(GRID,),
        in_specs=[
            pl.BlockSpec((2, RBP, D_OUT), lambda i: (0, i, 0)),
            pl.BlockSpec((RBP, D_OUT), lambda i: (i, 0)),
            pl.BlockSpec((RBP, 1), lambda i: (i, 0)),
            pl.BlockSpec((1, D_OUT), lambda i: (0, 0)),
        ],
        out_specs=pl.BlockSpec((RBP, D_OUT), lambda i: (i, 0)),
        out_shape=jax.ShapeDtypeStruct((N, D_OUT), jnp.float32),
    )(q, g2, dinv, b2.reshape(1, D_OUT))


# speedup vs baseline: 1.0462x; 1.0319x over previous
import functools

import jax
import jax.numpy as jnp
from jax import lax
from jax.experimental import pallas as pl
from jax.experimental.pallas import tpu as pltpu
from jax.experimental.pallas import tpu_sc as plsc

N = 10000
E = 320000
D_IN = 128
D_H = 64
D_OUT = 16
SLOPE_NEG = 0.01

NPAD = 10240
ROWS_PER_TILE = NPAD // 16
NC = 2
NS = 16
KB = 128
BPT = 80
NBLK = E // KB
TAIL = NBLK - (NC * NS - 1) * BPT
NBUF = 8
NG = BPT // NBUF
_RING_CFG = {64: (8, 6), 16: (10, 8)}


def _mesh():
    return plsc.VectorSubcoreMesh(core_axis_name="c", subcore_axis_name="s")


def _degree_body(dst_hbm, out_hbm, didx, ones_v, zbuf, acc, dsem):
    cid = lax.axis_index("c")
    sid = lax.axis_index("s")
    wid = cid * NS + sid
    base_row = sid * ROWS_PER_TILE

    ov = jnp.ones((16,), jnp.float32)
    zv = jnp.zeros((16,), jnp.float32)
    for i in range(KB // 16):
        ones_v[pl.ds(i * 16, 16)] = ov
    for i in range(ROWS_PER_TILE // 16):
        zbuf[pl.ds(i * 16, 16)] = zv
    base_blk = wid * BPT
    nblk = jnp.minimum(BPT, NBLK - base_blk)

    @pl.when(base_blk + BPT <= NBLK)
    def _():
        pltpu.sync_copy(dst_hbm.at[pl.ds(base_blk, BPT)], didx)

    @pl.when(base_blk + BPT > NBLK)
    def _():
        pltpu.sync_copy(dst_hbm.at[pl.ds(base_blk, TAIL)],
                        didx.at[pl.ds(0, TAIL)])

    pltpu.sync_copy(zbuf, acc.at[pl.ds(base_row, ROWS_PER_TILE)])
    plsc.subcore_barrier()

    @pl.loop(0, nblk)
    def _(j):
        pltpu.async_copy(ones_v, acc.at[didx.at[j]], dsem, add=True)

    @pl.loop(0, nblk)
    def _(j):
        pltpu.make_async_copy(ones_v, acc.at[pl.ds(0, KB)], dsem).wait()

    plsc.subcore_barrier()
    pltpu.sync_copy(acc.at[pl.ds(base_row, ROWS_PER_TILE)],
                    out_hbm.at[cid, pl.ds(base_row, ROWS_PER_TILE)])


_degree = pl.kernel(
    _degree_body,
    out_type=jax.ShapeDtypeStruct((NC, NPAD), jnp.float32),
    mesh=_mesh(),
    compiler_params=pltpu.CompilerParams(use_tc_tiling_on_sc=False),
    scratch_types=[
        pltpu.VMEM((BPT, KB), jnp.int32),
        pltpu.VMEM((KB,), jnp.float32),
        pltpu.VMEM((ROWS_PER_TILE,), jnp.float32),
        pltpu.VMEM_SHARED((NPAD,), jnp.float32),
        pltpu.SemaphoreType.DMA,
    ],
)


def _agg_body(F, g_hbm, src_hbm, dst_hbm, out_hbm, sidx, didx, rows, ztile,
              acc, gsem, ssem, zsem):
    RING, LOOK = _RING_CFG[F]
    cid = lax.axis_index("c")
    sid = lax.axis_index("s")
    wid = cid * NS + sid
    base_row = sid * ROWS_PER_TILE

    zv = jnp.zeros((16,), jnp.float32)
    for r in range(16):
        for c in range(F // 16):
            ztile[r, pl.ds(c * 16, 16)] = zv

    nz = ROWS_PER_TILE // 16

    @pl.loop(0, nz)
    def _(r):
        pltpu.async_copy(ztile, acc.at[pl.ds(base_row + r * 16, 16)], zsem)

    base_blk = wid * BPT

    @pl.when(base_blk + BPT <= NBLK)
    def _():
        pltpu.sync_copy(src_hbm.at[pl.ds(base_blk, BPT)], sidx)
        pltpu.sync_copy(dst_hbm.at[pl.ds(base_blk, BPT)], didx)

    @pl.when(base_blk + BPT > NBLK)
    def _():
        pltpu.sync_copy(src_hbm.at[pl.ds(base_blk, TAIL)],
                        sidx.at[pl.ds(0, TAIL)])
        pltpu.sync_copy(dst_hbm.at[pl.ds(base_blk, TAIL)],
                        didx.at[pl.ds(0, TAIL)])

    def gwait(b):
        pltpu.make_async_copy(g_hbm.at[sidx.at[0]], rows.at[b],
                              gsem.at[b]).wait()

    def swait(b):
        pltpu.make_async_copy(rows.at[b], acc.at[didx.at[0]],
                              ssem.at[b]).wait()

    @pl.loop(0, nz)
    def _(r):
        pltpu.make_async_copy(ztile, acc.at[pl.ds(base_row, 16)], zsem).wait()

    plsc.subcore_barrier()

    for c in range(LOOK):
        @pl.when(base_blk + c < NBLK)
        def _(c=c):
            pltpu.async_copy(g_hbm.at[sidx.at[c]], rows.at[c], gsem.at[c])

    def visit(c, b, do_swait, do_refill):
        @pl.when(base_blk + c < NBLK)
        def _():
            gwait(b)
            pltpu.async_copy(rows.at[b], acc.at[didx.at[c]],
                             ssem.at[b], add=True)
        if do_refill:
            b2 = (b + LOOK) % RING

            @pl.when(base_blk + c + LOOK < NBLK)
            def _():
                if do_swait:
                    swait(b2)
                pltpu.async_copy(g_hbm.at[sidx.at[c + LOOK]], rows.at[b2],
                                 gsem.at[b2])

    for b in range(RING):
        visit(b, b, b >= RING - LOOK, True)

    @pl.loop(1, BPT // RING - 1)
    def _(g):
        c0 = g * RING
        for b in range(RING):
            visit(c0 + b, b, True, True)

    for b in range(RING):
        visit(BPT - RING + b, b, True, b < RING - LOOK)
    for b in range(RING):
        swait(b)

    plsc.subcore_barrier()
    pltpu.sync_copy(acc.at[pl.ds(base_row, ROWS_PER_TILE)],
                    out_hbm.at[cid, pl.ds(base_row, ROWS_PER_TILE)])


def _make_agg(F):
    RING, _ = _RING_CFG[F]
    return pl.kernel(
        functools.partial(_agg_body, F),
        out_type=jax.ShapeDtypeStruct((NC, NPAD, F), jnp.float32),
        mesh=_mesh(),
        compiler_params=pltpu.CompilerParams(use_tc_tiling_on_sc=False),
        scratch_types=[
            pltpu.VMEM((BPT, KB), jnp.int32),
            pltpu.VMEM((BPT, KB), jnp.int32),
            pltpu.VMEM((RING, KB, F), jnp.float32),
            pltpu.VMEM((16, F), jnp.float32),
            pltpu.VMEM_SHARED((NPAD, F), jnp.float32),
            pltpu.SemaphoreType.DMA((RING,)),
            pltpu.SemaphoreType.DMA((RING,)),
            pltpu.SemaphoreType.DMA,
        ],
    )


_agg64 = _make_agg(D_H)
_agg16 = _make_agg(D_OUT)


def _tc_first(x_ref, w_ref, d0_ref, d1_ref, g_ref, dinv_ref):
    deg = d0_ref[...] + d1_ref[...] + 1.0
    dinv = lax.rsqrt(deg)
    dinv_ref[...] = dinv
    h = jnp.dot(x_ref[...], w_ref[...], preferred_element_type=jnp.float32)
    g_ref[pl.ds(0, N), :] = h * dinv[0:N]


def _tc_mid(p_ref, g1_ref, dinv_ref, b1_ref, w2_ref, g2_ref):
    dinv = dinv_ref[...]
    t = (p_ref[0] + p_ref[1] + g1_ref[...]) * dinv + b1_ref[...]
    z = jnp.where(t >= 0.0, t, SLOPE_NEG * t)
    g2_ref[...] = jnp.dot(z, w2_ref[...],
                          preferred_element_type=jnp.float32) * dinv


def _tc_final(q_ref, g2_ref, dinv_ref, b2_ref, o_ref):
    full = ((q_ref[0] + q_ref[1] + g2_ref[...]) * dinv_ref[...]
            + b2_ref[...])
    o_ref[...] = full[0:N, :]


def kernel(x, edge_index, W1, b1, W2, b2):
    src2 = edge_index[0].reshape(NBLK, KB)
    dst2 = edge_index[1].reshape(NBLK, KB)

    degp = _degree(dst2)

    g1, dinv = pl.pallas_call(
        _tc_first,
        out_shape=(jax.ShapeDtypeStruct((NPAD, D_H), jnp.float32),
                   jax.ShapeDtypeStruct((NPAD, 1), jnp.float32)),
    )(x, W1, degp[0].reshape(NPAD, 1), degp[1].reshape(NPAD, 1))

    p = _agg64(g1, src2, dst2)

    g2 = pl.pallas_call(
        _tc_mid,
        out_shape=jax.ShapeDtypeStruct((NPAD, D_OUT), jnp.float32),
    )(p, g1, dinv, b1.reshape(1, D_H), W2)

    q = _agg16(g2, src2, dst2)

    return pl.pallas_call(
        _tc_final,
        out_shape=jax.ShapeDtypeStruct((N, D_OUT), jnp.float32),
    )(q, g2, dinv, b2.reshape(1, D_OUT))
